# baseline (speedup 1.0000x reference)
import functools

import jax
import jax.numpy as jnp
from jax import lax
from jax.experimental import pallas as pl
from jax.experimental.pallas import tpu as pltpu
from jax.experimental.pallas import tpu_sc as plsc

NC = 2
NS = 16
NW = NC * NS
LANES = 16


@functools.lru_cache(maxsize=None)
def _build(B, L, V, D):
    EPW = B // NW
    TPW = EPW * L
    DV = D // LANES
    CH0 = 104
    CH1 = L - CH0
    NBUF = 2
    PD = 2 * D

    mesh = plsc.VectorSubcoreMesh(core_axis_name="c", subcore_axis_name="s")

    @functools.partial(
        pl.kernel,
        out_type=jax.ShapeDtypeStruct((B, D), jnp.float32),
        mesh=mesh,
        compiler_params=pltpu.CompilerParams(use_tc_tiling_on_sc=False),
        scratch_types=[
            pltpu.VMEM((TPW + LANES,), jnp.int32),
            pltpu.VMEM((TPW,), jnp.int32),
            pltpu.VMEM((NBUF, L, PD), jnp.float32),
            pltpu.VMEM((EPW, D), jnp.float32),
            [pltpu.SemaphoreType.DMA] * NBUF,
        ],
    )
    def encoder(tok_hbm, table_hbm, out_hbm, tok_v, idx_v, rows_v, out_v, sems):
        wid = lax.axis_index("s") * NC + lax.axis_index("c")
        base = wid * EPW

        pltpu.sync_copy(tok_hbm.at[wid], tok_v.at[pl.ds(0, TPW)])

        def prep(k, carry):
            t = tok_v[pl.ds(k * LANES, LANES)]
            idx_v[pl.ds(k * LANES, LANES)] = lax.shift_right_logical(t, 1)
            tok_v[pl.ds(k * LANES, LANES)] = lax.bitwise_and(t, 1)
            return carry

        lax.fori_loop(0, TPW // LANES, prep, 0, unroll=8)

        def fire(e, b):
            pltpu.async_copy(
                table_hbm.at[idx_v.at[pl.ds(e * L, CH0)]],
                rows_v.at[b, pl.ds(0, CH0)],
                sems[b],
            )
            pltpu.async_copy(
                table_hbm.at[idx_v.at[pl.ds(e * L + CH0, CH1)]],
                rows_v.at[b, pl.ds(CH0, CH1)],
                sems[b],
            )

        def drain(e, b):
            pltpu.make_async_copy(
                table_hbm.at[idx_v.at[pl.ds(e * L, CH0)]],
                rows_v.at[b, pl.ds(0, CH0)],
                sems[b],
            ).wait()
            pltpu.make_async_copy(
                table_hbm.at[idx_v.at[pl.ds(e * L + CH0, CH1)]],
                rows_v.at[b, pl.ds(CH0, CH1)],
                sems[b],
            ).wait()

        for b in range(NBUF):
            fire(b, b)

        inv_l = jnp.float32(1.0 / L)

        NG = L // LANES
        TAIL = L - NG * LANES

        def reduce_elem(e, b):
            def group(g, accs, cnt):
                hv = tok_v[pl.ds(e * L + g * LANES, LANES)]
                for jj in range(cnt):
                    half = hv[jj] * D
                    j = g * LANES + jj
                    accs = tuple(
                        a + rows_v[b, j, pl.ds(half + k * LANES, LANES)]
                        for k, a in enumerate(accs)
                    )
                return accs

            init = tuple(jnp.zeros((LANES,), jnp.float32) for _ in range(DV))
            accs = lax.fori_loop(
                0, NG, lambda g, accs: group(g, accs, LANES), init,
            )
            if TAIL:
                accs = group(NG, accs, TAIL)
            for k in range(DV):
                out_v[e, pl.ds(k * LANES, LANES)] = accs[k] * inv_l

        def outer(g, carry):
            for b in range(NBUF):
                e = g * NBUF + b
                drain(e, b)
                reduce_elem(e, b)

                @pl.when(e + NBUF < EPW)
                def _():
                    fire(e + NBUF, b)
            return carry

        lax.fori_loop(0, EPW // NBUF, outer, 0)

        pltpu.sync_copy(out_v, out_hbm.at[pl.ds(base, EPW)])

    return encoder


def kernel(token_ids, table):
    B, L = token_ids.shape
    V, D = table.shape
    enc = _build(B, L, V, D)
    tok = token_ids.astype(jnp.int32).reshape(NW, (B // NW) * L)
    pair_table = table.reshape(V // 2, 2 * D)
    return enc(tok, pair_table)


# speedup vs baseline: 1.0005x; 1.0005x over previous
import functools

import jax
import jax.numpy as jnp
from jax import lax
from jax.experimental import pallas as pl
from jax.experimental.pallas import tpu as pltpu
from jax.experimental.pallas import tpu_sc as plsc

NC = 2
NS = 16
NW = NC * NS
LANES = 16


@functools.lru_cache(maxsize=None)
def _build(B, L, V, D):
    EPW = B // NW
    TPW = EPW * L
    DV = D // LANES
    CH0 = 104
    CH1 = L - CH0
    NBUF = 2
    PD = 2 * D

    mesh = plsc.VectorSubcoreMesh(core_axis_name="c", subcore_axis_name="s")

    @functools.partial(
        pl.kernel,
        out_type=jax.ShapeDtypeStruct((B, D), jnp.float32),
        mesh=mesh,
        compiler_params=pltpu.CompilerParams(use_tc_tiling_on_sc=True),
        scratch_types=[
            pltpu.VMEM((TPW + LANES,), jnp.int32),
            pltpu.VMEM((TPW,), jnp.int32),
            pltpu.VMEM((NBUF, L, PD), jnp.float32),
            pltpu.VMEM((EPW, D), jnp.float32),
            [pltpu.SemaphoreType.DMA] * NBUF,
        ],
    )
    def encoder(tok_hbm, table_hbm, out_hbm, tok_v, idx_v, rows_v, out_v, sems):
        wid = lax.axis_index("s") * NC + lax.axis_index("c")
        base = wid * EPW

        pltpu.sync_copy(tok_hbm.at[wid], tok_v.at[pl.ds(0, TPW)])

        def prep(k, carry):
            t = tok_v[pl.ds(k * LANES, LANES)]
            idx_v[pl.ds(k * LANES, LANES)] = lax.shift_right_logical(t, 1)
            tok_v[pl.ds(k * LANES, LANES)] = lax.bitwise_and(t, 1)
            return carry

        lax.fori_loop(0, TPW // LANES, prep, 0, unroll=8)

        def fire(e, b):
            pltpu.async_copy(
                table_hbm.at[idx_v.at[pl.ds(e * L, CH0)]],
                rows_v.at[b, pl.ds(0, CH0)],
                sems[b],
            )
            pltpu.async_copy(
                table_hbm.at[idx_v.at[pl.ds(e * L + CH0, CH1)]],
                rows_v.at[b, pl.ds(CH0, CH1)],
                sems[b],
            )

        def drain(e, b):
            pltpu.make_async_copy(
                table_hbm.at[idx_v.at[pl.ds(e * L, CH0)]],
                rows_v.at[b, pl.ds(0, CH0)],
                sems[b],
            ).wait()
            pltpu.make_async_copy(
                table_hbm.at[idx_v.at[pl.ds(e * L + CH0, CH1)]],
                rows_v.at[b, pl.ds(CH0, CH1)],
                sems[b],
            ).wait()

        for b in range(NBUF):
            fire(b, b)

        inv_l = jnp.float32(1.0 / L)

        NG = L // LANES
        TAIL = L - NG * LANES

        def reduce_elem(e, b):
            def group(g, accs, cnt):
                hv = tok_v[pl.ds(e * L + g * LANES, LANES)]
                for jj in range(cnt):
                    half = hv[jj] * D
                    j = g * LANES + jj
                    accs = tuple(
                        a + rows_v[b, j, pl.ds(half + k * LANES, LANES)]
                        for k, a in enumerate(accs)
                    )
                return accs

            init = tuple(jnp.zeros((LANES,), jnp.float32) for _ in range(DV))
            accs = lax.fori_loop(
                0, NG, lambda g, accs: group(g, accs, LANES), init,
            )
            if TAIL:
                accs = group(NG, accs, TAIL)
            for k in range(DV):
                out_v[e, pl.ds(k * LANES, LANES)] = accs[k] * inv_l

        def outer(g, carry):
            for b in range(NBUF):
                e = g * NBUF + b
                drain(e, b)
                reduce_elem(e, b)

                @pl.when(e + NBUF < EPW)
                def _():
                    fire(e + NBUF, b)
            return carry

        lax.fori_loop(0, EPW // NBUF, outer, 0)

        pltpu.sync_copy(out_v, out_hbm.at[pl.ds(base, EPW)])

    return encoder


def kernel(token_ids, table):
    B, L = token_ids.shape
    V, D = table.shape
    enc = _build(B, L, V, D)
    tok = token_ids.astype(jnp.int32).reshape(NW, (B // NW) * L)
    pair_table = table.reshape(V // 2, 2 * D)
    return enc(tok, pair_table)


# speedup vs baseline: 1.1896x; 1.1890x over previous
import functools

import jax
import jax.numpy as jnp
from jax import lax
from jax.experimental import pallas as pl
from jax.experimental.pallas import tpu as pltpu
from jax.experimental.pallas import tpu_sc as plsc

NC = 2
NS = 16
NW = NC * NS
LANES = 16


@functools.lru_cache(maxsize=None)
def _build(B, L, V, D):
    EPW = B // NW
    NCH = -(-L // 128)
    assert L % NCH == 0
    CH = L // NCH
    DV = D // LANES
    NBUF = 4
    ROWS_PER_W = EPW * NCH

    mesh = plsc.VectorSubcoreMesh(core_axis_name="c", subcore_axis_name="s")

    @functools.partial(
        pl.kernel,
        out_type=jax.ShapeDtypeStruct((B, D), jnp.float32),
        mesh=mesh,
        compiler_params=pltpu.CompilerParams(use_tc_tiling_on_sc=False),
        scratch_types=[
            pltpu.VMEM((ROWS_PER_W, CH), jnp.int32),
            pltpu.VMEM((NBUF, L, D), jnp.float32),
            pltpu.VMEM((EPW, D), jnp.float32),
            [pltpu.SemaphoreType.DMA] * NBUF,
        ],
    )
    def encoder(tok_hbm, table_hbm, out_hbm, idx_v, rows_v, out_v, sems):
        wid = lax.axis_index("s") * NC + lax.axis_index("c")
        base = wid * EPW

        pltpu.sync_copy(tok_hbm.at[pl.ds(wid * ROWS_PER_W, ROWS_PER_W)], idx_v)

        def fire(e, b):
            for c in range(NCH):
                pltpu.async_copy(
                    table_hbm.at[idx_v.at[e * NCH + c]],
                    rows_v.at[b, pl.ds(c * CH, CH)],
                    sems[b],
                )

        def drain(e, b):
            for c in range(NCH):
                pltpu.make_async_copy(
                    table_hbm.at[idx_v.at[e * NCH + c]],
                    rows_v.at[b, pl.ds(c * CH, CH)],
                    sems[b],
                ).wait()

        for b in range(NBUF):
            fire(b, b)

        inv_l = jnp.float32(1.0 / L)

        def reduce_elem(e, b):
            def body(j, accs):
                return tuple(
                    a + rows_v[b, j, pl.ds(k * LANES, LANES)]
                    for k, a in enumerate(accs)
                )
            accs = lax.fori_loop(
                0, L, body,
                tuple(jnp.zeros((LANES,), jnp.float32) for _ in range(DV)),
                unroll=8,
            )
            for k in range(DV):
                out_v[e, pl.ds(k * LANES, LANES)] = accs[k] * inv_l

        def outer(g, carry):
            for b in range(NBUF):
                e = g * NBUF + b
                drain(e, b)
                reduce_elem(e, b)

                @pl.when(e + NBUF < EPW)
                def _():
                    fire(e + NBUF, b)
            return carry

        lax.fori_loop(0, EPW // NBUF, outer, 0)

        pltpu.sync_copy(out_v, out_hbm.at[pl.ds(base, EPW)])

    return encoder


def kernel(token_ids, table):
    B, L = token_ids.shape
    V, D = table.shape
    enc = _build(B, L, V, D)
    NCH = -(-L // 128)
    tok = token_ids.astype(jnp.int32).reshape(B * NCH, L // NCH)
    table_rm = lax.optimization_barrier(table.T).T
    return enc(tok, table_rm)
